# baseline (speedup 1.0000x reference)
import functools

import jax
import jax.numpy as jnp
from jax import lax
from jax.experimental import pallas as pl
from jax.experimental.pallas import tpu as pltpu
from jax.experimental.pallas import tpu_sc as plsc

NC = 2
NS = 16
NW = NC * NS
ROW = 128
CHUNK_ROWS = 2
CNT_CHUNK = 4


def _edge_mesh():
    return plsc.VectorSubcoreMesh(core_axis_name="c", subcore_axis_name="s")


def _make_edge_pass(acc_rows, hid, rows_pad):
    rows_per_worker = rows_pad // NS
    n_chunks = rows_per_worker // CHUNK_ROWS
    assert n_chunks % 2 == 0 and n_chunks >= 4
    rows_per_sub = acc_rows // NS
    cw = CHUNK_ROWS * ROW

    @functools.partial(
        pl.kernel,
        out_type=jax.ShapeDtypeStruct((acc_rows, hid), jnp.float32),
        mesh=_edge_mesh(),
        scratch_types=[
            pltpu.VMEM((2, 2 * CHUNK_ROWS, ROW), jnp.int32),
            pltpu.VMEM((2, cw, hid), jnp.float32),
            pltpu.VMEM_SHARED((acc_rows, hid), jnp.float32),
            pltpu.SemaphoreType.DMA,
            pltpu.SemaphoreType.DMA,
            pltpu.SemaphoreType.DMA,
            pltpu.SemaphoreType.DMA,
            pltpu.SemaphoreType.DMA,
            pltpu.SemaphoreType.DMA,
            pltpu.SemaphoreType.DMA,
        ],
        compiler_params=pltpu.CompilerParams(use_tc_tiling_on_sc=False),
    )
    def edge_pass(h3_hbm, idx_hbm, zeros_hbm, out_hbm,
                  idx_v, msg_v, acc_sh,
                  gsem0, gsem1, ssem0, ssem1, isem0, isem1, zsem):
        core = lax.axis_index("c")
        sub = lax.axis_index("s")
        gsem = (gsem0, gsem1)
        ssem = (ssem0, ssem1)
        isem = (isem0, isem1)

        @pl.when(core == 0)
        def _sc0():
            my_acc = pl.ds(sub * rows_per_sub, rows_per_sub)
            zcp = pltpu.async_copy(zeros_hbm.at[my_acc], acc_sh.at[my_acc],
                                   zsem)
            row0 = sub * rows_per_worker

            def load_idx(i, s):
                return pltpu.async_copy(
                    idx_hbm.at[pl.ds(2 * (row0 + i * CHUNK_ROWS),
                                     2 * CHUNK_ROWS)],
                    idx_v.at[s], isem[s])

            def issue_gathers(i, s):
                for j in range(CHUNK_ROWS):
                    pltpu.async_copy(h3_hbm.at[idx_v.at[s].at[2 * j]],
                                     msg_v.at[s].at[pl.ds(j * ROW, ROW)],
                                     gsem[s])

            def issue_scatters(i, s):
                for j in range(CHUNK_ROWS):
                    pltpu.async_copy(msg_v.at[s].at[pl.ds(j * ROW, ROW)],
                                     acc_sh.at[idx_v.at[s].at[2 * j + 1]],
                                     ssem[s], add=True)

            def drain(sem, s):
                pltpu.make_async_copy(h3_hbm.at[pl.ds(0, cw)], msg_v.at[s],
                                      sem).wait()

            def drain_idx(s):
                pltpu.make_async_copy(idx_hbm.at[pl.ds(0, 2 * CHUNK_ROWS)],
                                      idx_v.at[s], isem[s]).wait()

            cp0 = load_idx(0, 0)
            cp1 = load_idx(1, 1)
            zcp.wait()
            plsc.subcore_barrier()
            cp0.wait()
            issue_gathers(0, 0)
            cp1.wait()
            issue_gathers(1, 1)

            @pl.loop(0, (n_chunks - 2) // 2)
            def _pair(k):
                i0 = 2 * k
                drain(gsem[0], 0)
                issue_scatters(i0, 0)
                drain(ssem[0], 0)
                load_idx(i0 + 2, 0)
                drain(gsem[1], 1)
                issue_scatters(i0 + 1, 1)
                drain(ssem[1], 1)
                load_idx(i0 + 3, 1)
                drain_idx(0)
                issue_gathers(i0 + 2, 0)
                drain_idx(1)
                issue_gathers(i0 + 3, 1)

            drain(gsem[0], 0)
            issue_scatters(n_chunks - 2, 0)
            drain(gsem[1], 1)
            issue_scatters(n_chunks - 1, 1)
            drain(ssem[0], 0)
            drain(ssem[1], 1)
            plsc.subcore_barrier()
            pltpu.sync_copy(acc_sh.at[my_acc], out_hbm.at[my_acc])

    return edge_pass


def _make_cnt_pass(cacc_rows, rows_pad):
    rows_per_worker = rows_pad // NW
    n_chunks = rows_per_worker // CNT_CHUNK
    rows_per_sub = cacc_rows // NS

    @functools.partial(
        pl.kernel,
        out_type=jax.ShapeDtypeStruct((NC, cacc_rows, 16), jnp.float32),
        mesh=_edge_mesh(),
        scratch_types=[
            pltpu.VMEM((CNT_CHUNK, ROW), jnp.int32),
            pltpu.VMEM((ROW, 16), jnp.float32),
            pltpu.VMEM_SHARED((cacc_rows, 16), jnp.float32),
        ],
        compiler_params=pltpu.CompilerParams(use_tc_tiling_on_sc=False),
    )
    def cnt_pass(cidx_hbm, ones_hbm, zeros_hbm, out_hbm,
                 cidx_v, ones_v, acc_sh):
        core = lax.axis_index("c")
        sub = lax.axis_index("s")
        wid = core * NS + sub

        my_acc = pl.ds(sub * rows_per_sub, rows_per_sub)
        pltpu.sync_copy(zeros_hbm.at[my_acc], acc_sh.at[my_acc])
        pltpu.sync_copy(ones_hbm, ones_v)
        plsc.subcore_barrier()

        row0 = wid * rows_per_worker

        @pl.loop(0, n_chunks)
        def _chunk(i):
            r0 = row0 + i * CNT_CHUNK
            pltpu.sync_copy(cidx_hbm.at[pl.ds(r0, CNT_CHUNK)], cidx_v)
            for j in range(CNT_CHUNK):
                pltpu.sync_copy(ones_v, acc_sh.at[cidx_v.at[j]], add=True)

        plsc.subcore_barrier()
        pltpu.sync_copy(acc_sh.at[my_acc], out_hbm.at[core].at[my_acc])

    return cnt_pass


def _prep_body(etg_ref, src_ref, dsts_ref, etc_ref, dstc_ref,
               idx_ref, cidx_ref, *, n_nodes):
    rows = etg_ref.shape[0]
    gidx = etg_ref[...] * n_nodes + src_ref[...]
    idx_ref[...] = jnp.stack([gidx, dsts_ref[...]], axis=1).reshape(
        2 * rows, gidx.shape[1])
    cidx_ref[...] = etc_ref[...] * n_nodes + dstc_ref[...]


def _proj_body(x_ref, w_ref, out_ref):
    x = x_ref[...]
    for c in range(w_ref.shape[0]):
        out_ref[c] = jnp.dot(x, w_ref[c], preferred_element_type=jnp.float32)


def _make_scale_body(c, n_rel, n_nodes):
    def body(h_ref, gt_ref, o_ref):
        f = jax.nn.softmax(gt_ref[...], axis=-1)[c]
        h = h_ref[...]
        for r in range(n_rel):
            o_ref[pl.ds(r * n_nodes, n_nodes)] = f[r] * h
    return body


def _make_combine_body(c, n_rel):
    def body(p_ref, c0_ref, c1_ref, c2_ref, h_ref, gt_ref, wg_ref, bg_ref,
             o_ref):
        f = jax.nn.softmax(gt_ref[...], axis=-1)[c]
        s = p_ref[...]
        c16 = (f[0] * c0_ref[...] + f[1] * c1_ref[...]
               + f[2] * c2_ref[...])
        deg = c16[0, :, 0:1] + c16[1, :, 0:1] + f[n_rel]
        h = h_ref[...]
        agg = (s + f[n_rel] * h) / deg
        o_ref[...] = jnp.maximum(
            jnp.dot(agg, wg_ref[...], preferred_element_type=jnp.float32)
            + bg_ref[...], 0.0)
    return body


def _head_body(x0_ref, x1_ref, w1_ref, b1_ref, w2_ref, b2_ref, y_ref):
    hid = x0_ref.shape[1]
    xw = (jnp.dot(x0_ref[...], w1_ref[pl.ds(0, hid)],
                  preferred_element_type=jnp.float32)
          + jnp.dot(x1_ref[...], w1_ref[pl.ds(hid, hid)],
                    preferred_element_type=jnp.float32))
    h = jnp.maximum(xw + b1_ref[...], 0.0)
    y_ref[...] = jnp.dot(h, w2_ref[...], preferred_element_type=jnp.float32) \
        + b2_ref[...]


def kernel(x, edge_index, etype, W_gcn, gt_weight, Wg, bg, W1, b1, W2, b2):
    n, in_dim = x.shape
    e = edge_index.shape[1]
    n_ch, _, hid = W_gcn.shape
    n_layer, _, n_rel_full = gt_weight.shape
    n_rel = n_rel_full - 1
    num_class = W2.shape[1]

    rows_e = e // ROW
    import math
    row_gran = math.lcm(NS * CHUNK_ROWS * 2, NW * CNT_CHUNK)
    rows_pad = ((rows_e + row_gran - 1) // row_gran) * row_gran
    pad_edges = rows_pad * ROW - e
    acc_rows = ((n + 16 + 127) // 128) * 128
    cacc_rows = ((n_rel * n + 16 + 127) // 128) * 128

    src = edge_index[0]
    dst = edge_index[1]
    padk = jnp.arange(pad_edges, dtype=jnp.int32) % 16
    zpad = jnp.zeros((pad_edges,), jnp.int32)
    et32 = etype.astype(jnp.int32)
    src_p = jnp.concatenate([src, zpad]).reshape(rows_pad, ROW)
    etg_p = jnp.concatenate([et32, zpad]).reshape(rows_pad, ROW)
    dsts_p = jnp.concatenate([dst, padk + n]).reshape(rows_pad, ROW)
    etc_p = jnp.concatenate(
        [et32, jnp.full((pad_edges,), n_rel, jnp.int32)]).reshape(rows_pad, ROW)
    dstc_p = jnp.concatenate([dst, padk]).reshape(rows_pad, ROW)
    zeros_acc = jnp.zeros((acc_rows, hid), jnp.float32)
    zeros_cnt = jnp.zeros((cacc_rows, 16), jnp.float32)
    ones_row = jnp.ones((ROW, 16), jnp.float32)

    idx_il, cidx = pl.pallas_call(
        functools.partial(_prep_body, n_nodes=n),
        out_shape=[jax.ShapeDtypeStruct((2 * rows_pad, ROW), jnp.int32),
                   jax.ShapeDtypeStruct((rows_pad, ROW), jnp.int32)],
    )(etg_p, src_p, dsts_p, etc_p, dstc_p)

    cnt_pass = _make_cnt_pass(cacc_rows, rows_pad)
    cnt_part = cnt_pass(cidx, ones_row, zeros_cnt)

    H0 = pl.pallas_call(
        _proj_body,
        out_shape=jax.ShapeDtypeStruct((n_ch, n, hid), jnp.float32),
    )(x, W_gcn)

    edge_pass = _make_edge_pass(acc_rows, hid, rows_pad)

    scale_calls = [
        pl.pallas_call(
            _make_scale_body(c, n_rel, n),
            out_shape=jax.ShapeDtypeStruct((n_rel * n, hid), jnp.float32),
        ) for c in range(n_ch)
    ]

    BLK = 2000
    assert n % BLK == 0 and n_rel == 3
    nb = n // BLK
    grid = (nb,)
    combine_calls = []
    for c in range(n_ch):
        combine_calls.append(pl.pallas_call(
            _make_combine_body(c, n_rel),
            grid=grid,
            in_specs=[
                pl.BlockSpec((BLK, hid), lambda i: (i, 0)),
                pl.BlockSpec((NC, BLK, 16), lambda i: (0, i, 0)),
                pl.BlockSpec((NC, BLK, 16), lambda i, _nb=nb: (0, _nb + i, 0)),
                pl.BlockSpec((NC, BLK, 16),
                             lambda i, _nb=nb: (0, 2 * _nb + i, 0)),
                pl.BlockSpec((BLK, hid), lambda i: (i, 0)),
                pl.BlockSpec((n_ch, n_rel_full), lambda i: (0, 0)),
                pl.BlockSpec((hid, hid), lambda i: (0, 0)),
                pl.BlockSpec((1, hid), lambda i: (0, 0)),
            ],
            out_specs=pl.BlockSpec((BLK, hid), lambda i: (i, 0)),
            out_shape=jax.ShapeDtypeStruct((n, hid), jnp.float32),
        ))

    H = [H0[c] for c in range(n_ch)]
    for l in range(n_layer):
        newH = []
        for c in range(n_ch):
            h3 = scale_calls[c](H[c], gt_weight[l])
            part = edge_pass(h3, idx_il, zeros_acc)
            newH.append(combine_calls[c](
                part, cnt_part, cnt_part, cnt_part, H[c], gt_weight[l],
                Wg, bg.reshape(1, hid)))
        H = newH

    y = pl.pallas_call(
        _head_body,
        out_shape=jax.ShapeDtypeStruct((n, num_class), jnp.float32),
    )(H[0], H[1], W1, b1.reshape(1, hid), W2, b2.reshape(1, num_class))
    return y


# speedup vs baseline: 1.3004x; 1.3004x over previous
import functools

import jax
import jax.numpy as jnp
from jax import lax
from jax.experimental import pallas as pl
from jax.experimental.pallas import tpu as pltpu
from jax.experimental.pallas import tpu_sc as plsc

NC = 2
NS = 16
NW = NC * NS
ROW = 128
CHUNK_ROWS = 2
CNT_CHUNK = 4


def _edge_mesh():
    return plsc.VectorSubcoreMesh(core_axis_name="c", subcore_axis_name="s")


def _make_edge_pass(acc_rows, hid, rows_pad):
    rows_per_pair = rows_pad // NS
    r1_rows = 2 * CHUNK_ROWS
    r0_rows = rows_per_pair - r1_rows
    assert r0_rows % (2 * CHUNK_ROWS) == 0 and r0_rows > 0
    rows_per_sub = acc_rows // NS
    cw = CHUNK_ROWS * ROW

    @functools.partial(
        pl.kernel,
        out_type=jax.ShapeDtypeStruct((NC, acc_rows, hid), jnp.float32),
        mesh=_edge_mesh(),
        scratch_types=[
            pltpu.VMEM((2 * r0_rows, ROW), jnp.int32),
            pltpu.VMEM((2, cw, hid), jnp.float32),
            pltpu.VMEM_SHARED((acc_rows, hid), jnp.float32),
            pltpu.SemaphoreType.DMA,
            pltpu.SemaphoreType.DMA,
            pltpu.SemaphoreType.DMA,
            pltpu.SemaphoreType.DMA,
            pltpu.SemaphoreType.DMA,
        ],
        compiler_params=pltpu.CompilerParams(use_tc_tiling_on_sc=False),
    )
    def edge_pass(h3_hbm, idx_hbm, zeros_hbm, out_hbm,
                  idx_v, msg_v, acc_sh,
                  gsem0, gsem1, ssem0, ssem1, zsem):
        core = lax.axis_index("c")
        sub = lax.axis_index("s")
        gsem = (gsem0, gsem1)
        ssem = (ssem0, ssem1)

        my_acc = pl.ds(sub * rows_per_sub, rows_per_sub)
        zcp = pltpu.async_copy(zeros_hbm.at[my_acc], acc_sh.at[my_acc], zsem)

        def issue_gathers(i, s):
            for j in range(CHUNK_ROWS):
                pltpu.async_copy(
                    h3_hbm.at[idx_v.at[2 * (i * CHUNK_ROWS + j)]],
                    msg_v.at[s].at[pl.ds(j * ROW, ROW)], gsem[s])

        def issue_scatters(i, s):
            for j in range(CHUNK_ROWS):
                pltpu.async_copy(
                    msg_v.at[s].at[pl.ds(j * ROW, ROW)],
                    acc_sh.at[idx_v.at[2 * (i * CHUNK_ROWS + j) + 1]],
                    ssem[s], add=True)

        def drain(sem, s):
            pltpu.make_async_copy(h3_hbm.at[pl.ds(0, cw)], msg_v.at[s],
                                  sem).wait()

        def run_worker(row0, rows):
            n_chunks = rows // CHUNK_ROWS
            pltpu.sync_copy(idx_hbm.at[pl.ds(2 * row0, 2 * rows)],
                            idx_v.at[pl.ds(0, 2 * rows)])
            issue_gathers(0, 0)
            issue_gathers(1, 1)

            @pl.loop(0, (n_chunks - 2) // 2)
            def _pair(k):
                i0 = 2 * k
                drain(gsem[0], 0)
                issue_scatters(i0, 0)
                drain(ssem[0], 0)
                issue_gathers(i0 + 2, 0)
                drain(gsem[1], 1)
                issue_scatters(i0 + 1, 1)
                drain(ssem[1], 1)
                issue_gathers(i0 + 3, 1)

            drain(gsem[0], 0)
            issue_scatters(n_chunks - 2, 0)
            drain(gsem[1], 1)
            issue_scatters(n_chunks - 1, 1)
            drain(ssem[0], 0)
            drain(ssem[1], 1)

        zcp.wait()
        plsc.subcore_barrier()

        @pl.when(core == 0)
        def _sc0():
            run_worker(sub * r0_rows, r0_rows)

        @pl.when(core == 1)
        def _sc1():
            run_worker(NS * r0_rows + sub * r1_rows, r1_rows)

        plsc.subcore_barrier()
        pltpu.sync_copy(acc_sh.at[my_acc], out_hbm.at[core].at[my_acc])

    return edge_pass


def _make_cnt_pass(cacc_rows, rows_pad):
    rows_per_worker = rows_pad // NW
    n_chunks = rows_per_worker // CNT_CHUNK
    rows_per_sub = cacc_rows // NS

    @functools.partial(
        pl.kernel,
        out_type=jax.ShapeDtypeStruct((NC, cacc_rows, 16), jnp.float32),
        mesh=_edge_mesh(),
        scratch_types=[
            pltpu.VMEM((CNT_CHUNK, ROW), jnp.int32),
            pltpu.VMEM((ROW, 16), jnp.float32),
            pltpu.VMEM_SHARED((cacc_rows, 16), jnp.float32),
        ],
        compiler_params=pltpu.CompilerParams(use_tc_tiling_on_sc=False),
    )
    def cnt_pass(cidx_hbm, ones_hbm, zeros_hbm, out_hbm,
                 cidx_v, ones_v, acc_sh):
        core = lax.axis_index("c")
        sub = lax.axis_index("s")
        wid = core * NS + sub

        my_acc = pl.ds(sub * rows_per_sub, rows_per_sub)
        pltpu.sync_copy(zeros_hbm.at[my_acc], acc_sh.at[my_acc])
        pltpu.sync_copy(ones_hbm, ones_v)
        plsc.subcore_barrier()

        row0 = wid * rows_per_worker

        @pl.loop(0, n_chunks)
        def _chunk(i):
            r0 = row0 + i * CNT_CHUNK
            pltpu.sync_copy(cidx_hbm.at[pl.ds(r0, CNT_CHUNK)], cidx_v)
            for j in range(CNT_CHUNK):
                pltpu.sync_copy(ones_v, acc_sh.at[cidx_v.at[j]], add=True)

        plsc.subcore_barrier()
        pltpu.sync_copy(acc_sh.at[my_acc], out_hbm.at[core].at[my_acc])

    return cnt_pass


def _prep_body(etg_ref, src_ref, dsts_ref, etc_ref, dstc_ref,
               idx_ref, cidx_ref, *, n_nodes):
    rows = etg_ref.shape[0]
    gidx = etg_ref[...] * n_nodes + src_ref[...]
    idx_ref[...] = jnp.stack([gidx, dsts_ref[...]], axis=1).reshape(
        2 * rows, gidx.shape[1])
    cidx_ref[...] = etc_ref[...] * n_nodes + dstc_ref[...]


def _proj_body(x_ref, w_ref, out_ref):
    x = x_ref[...]
    for c in range(w_ref.shape[0]):
        out_ref[c] = jnp.dot(x, w_ref[c], preferred_element_type=jnp.float32)


def _make_scale_body(c, n_rel, n_nodes):
    def body(h_ref, gt_ref, o_ref):
        f = jax.nn.softmax(gt_ref[...], axis=-1)[c]
        h = h_ref[...]
        for r in range(n_rel):
            o_ref[pl.ds(r * n_nodes, n_nodes)] = f[r] * h
    return body


def _make_combine_body(c, n_rel):
    def body(p_ref, c0_ref, c1_ref, c2_ref, h_ref, gt_ref, wg_ref, bg_ref,
             o_ref):
        f = jax.nn.softmax(gt_ref[...], axis=-1)[c]
        p = p_ref[...]
        s = p[0] + p[1]
        c16 = (f[0] * c0_ref[...] + f[1] * c1_ref[...]
               + f[2] * c2_ref[...])
        deg = c16[0, :, 0:1] + c16[1, :, 0:1] + f[n_rel]
        h = h_ref[...]
        agg = (s + f[n_rel] * h) / deg
        o_ref[...] = jnp.maximum(
            jnp.dot(agg, wg_ref[...], preferred_element_type=jnp.float32)
            + bg_ref[...], 0.0)
    return body


def _head_body(x0_ref, x1_ref, w1_ref, b1_ref, w2_ref, b2_ref, y_ref):
    hid = x0_ref.shape[1]
    xw = (jnp.dot(x0_ref[...], w1_ref[pl.ds(0, hid)],
                  preferred_element_type=jnp.float32)
          + jnp.dot(x1_ref[...], w1_ref[pl.ds(hid, hid)],
                    preferred_element_type=jnp.float32))
    h = jnp.maximum(xw + b1_ref[...], 0.0)
    y_ref[...] = jnp.dot(h, w2_ref[...], preferred_element_type=jnp.float32) \
        + b2_ref[...]


def kernel(x, edge_index, etype, W_gcn, gt_weight, Wg, bg, W1, b1, W2, b2):
    n, in_dim = x.shape
    e = edge_index.shape[1]
    n_ch, _, hid = W_gcn.shape
    n_layer, _, n_rel_full = gt_weight.shape
    n_rel = n_rel_full - 1
    num_class = W2.shape[1]

    rows_e = e // ROW
    import math
    row_gran = math.lcm(NS * CHUNK_ROWS * 2, NW * CNT_CHUNK)
    rows_pad = ((rows_e + row_gran - 1) // row_gran) * row_gran
    pad_edges = rows_pad * ROW - e
    acc_rows = ((n + 16 + 127) // 128) * 128
    cacc_rows = ((n_rel * n + 16 + 127) // 128) * 128

    src = edge_index[0]
    dst = edge_index[1]
    padk = jnp.arange(pad_edges, dtype=jnp.int32) % 16
    zpad = jnp.zeros((pad_edges,), jnp.int32)
    et32 = etype.astype(jnp.int32)
    src_p = jnp.concatenate([src, zpad]).reshape(rows_pad, ROW)
    etg_p = jnp.concatenate([et32, zpad]).reshape(rows_pad, ROW)
    dsts_p = jnp.concatenate([dst, padk + n]).reshape(rows_pad, ROW)
    etc_p = jnp.concatenate(
        [et32, jnp.full((pad_edges,), n_rel, jnp.int32)]).reshape(rows_pad, ROW)
    dstc_p = jnp.concatenate([dst, padk]).reshape(rows_pad, ROW)
    zeros_acc = jnp.zeros((acc_rows, hid), jnp.float32)
    zeros_cnt = jnp.zeros((cacc_rows, 16), jnp.float32)
    ones_row = jnp.ones((ROW, 16), jnp.float32)

    idx_il, cidx = pl.pallas_call(
        functools.partial(_prep_body, n_nodes=n),
        out_shape=[jax.ShapeDtypeStruct((2 * rows_pad, ROW), jnp.int32),
                   jax.ShapeDtypeStruct((rows_pad, ROW), jnp.int32)],
    )(etg_p, src_p, dsts_p, etc_p, dstc_p)

    cnt_pass = _make_cnt_pass(cacc_rows, rows_pad)
    cnt_part = cnt_pass(cidx, ones_row, zeros_cnt)

    H0 = pl.pallas_call(
        _proj_body,
        out_shape=jax.ShapeDtypeStruct((n_ch, n, hid), jnp.float32),
    )(x, W_gcn)

    edge_pass = _make_edge_pass(acc_rows, hid, rows_pad)

    scale_calls = [
        pl.pallas_call(
            _make_scale_body(c, n_rel, n),
            out_shape=jax.ShapeDtypeStruct((n_rel * n, hid), jnp.float32),
        ) for c in range(n_ch)
    ]

    BLK = 2000
    assert n % BLK == 0 and n_rel == 3
    nb = n // BLK
    grid = (nb,)
    combine_calls = []
    for c in range(n_ch):
        combine_calls.append(pl.pallas_call(
            _make_combine_body(c, n_rel),
            grid=grid,
            in_specs=[
                pl.BlockSpec((NC, BLK, hid), lambda i: (0, i, 0)),
                pl.BlockSpec((NC, BLK, 16), lambda i: (0, i, 0)),
                pl.BlockSpec((NC, BLK, 16), lambda i, _nb=nb: (0, _nb + i, 0)),
                pl.BlockSpec((NC, BLK, 16),
                             lambda i, _nb=nb: (0, 2 * _nb + i, 0)),
                pl.BlockSpec((BLK, hid), lambda i: (i, 0)),
                pl.BlockSpec((n_ch, n_rel_full), lambda i: (0, 0)),
                pl.BlockSpec((hid, hid), lambda i: (0, 0)),
                pl.BlockSpec((1, hid), lambda i: (0, 0)),
            ],
            out_specs=pl.BlockSpec((BLK, hid), lambda i: (i, 0)),
            out_shape=jax.ShapeDtypeStruct((n, hid), jnp.float32),
        ))

    H = [H0[c] for c in range(n_ch)]
    for l in range(n_layer):
        newH = []
        for c in range(n_ch):
            h3 = scale_calls[c](H[c], gt_weight[l])
            part = edge_pass(h3, idx_il, zeros_acc)
            newH.append(combine_calls[c](
                part, cnt_part, cnt_part, cnt_part, H[c], gt_weight[l],
                Wg, bg.reshape(1, hid)))
        H = newH

    y = pl.pallas_call(
        _head_body,
        out_shape=jax.ShapeDtypeStruct((n, num_class), jnp.float32),
    )(H[0], H[1], W1, b1.reshape(1, hid), W2, b2.reshape(1, num_class))
    return y


# speedup vs baseline: 3.1438x; 2.4175x over previous
import functools

import jax
import jax.numpy as jnp
from jax import lax
from jax.experimental import pallas as pl
from jax.experimental.pallas import tpu as pltpu
from jax.experimental.pallas import tpu_sc as plsc

NC = 2
NS = 16
NW = NC * NS
ROW = 128
CHUNK_ROWS = 2
CNT_CHUNK = 4


def _edge_mesh():
    return plsc.VectorSubcoreMesh(core_axis_name="c", subcore_axis_name="s")


def _make_edge_pass(acc_rows, hid, rows_pad):
    rows_per_pair = rows_pad // NS
    r1_rows = rows_per_pair // 2
    r0_rows = rows_per_pair - r1_rows
    assert r0_rows % (2 * CHUNK_ROWS) == 0 and r1_rows % (2 * CHUNK_ROWS) == 0
    rows_per_sub = acc_rows // NS
    cw = CHUNK_ROWS * ROW

    @functools.partial(
        pl.kernel,
        out_type=jax.ShapeDtypeStruct((NC, acc_rows, hid), jnp.float32),
        mesh=_edge_mesh(),
        scratch_types=[
            pltpu.VMEM((2 * r0_rows, ROW), jnp.int32),
            pltpu.VMEM((2, cw, hid), jnp.float32),
            pltpu.VMEM_SHARED((acc_rows, hid), jnp.float32),
            pltpu.SemaphoreType.DMA,
            pltpu.SemaphoreType.DMA,
            pltpu.SemaphoreType.DMA,
            pltpu.SemaphoreType.DMA,
            pltpu.SemaphoreType.DMA,
        ],
        compiler_params=pltpu.CompilerParams(use_tc_tiling_on_sc=False),
    )
    def edge_pass(h3_hbm, idx_hbm, zeros_hbm, out_hbm,
                  idx_v, msg_v, acc_sh,
                  gsem0, gsem1, ssem0, ssem1, zsem):
        core = lax.axis_index("c")
        sub = lax.axis_index("s")
        gsem = (gsem0, gsem1)
        ssem = (ssem0, ssem1)

        my_acc = pl.ds(sub * rows_per_sub, rows_per_sub)
        zcp = pltpu.async_copy(zeros_hbm.at[my_acc], acc_sh.at[my_acc], zsem)

        def issue_gathers(i, s):
            for j in range(CHUNK_ROWS):
                pltpu.async_copy(
                    h3_hbm.at[idx_v.at[2 * (i * CHUNK_ROWS + j)]],
                    msg_v.at[s].at[pl.ds(j * ROW, ROW)], gsem[s])

        def issue_scatters(i, s):
            for j in range(CHUNK_ROWS):
                pltpu.async_copy(
                    msg_v.at[s].at[pl.ds(j * ROW, ROW)],
                    acc_sh.at[idx_v.at[2 * (i * CHUNK_ROWS + j) + 1]],
                    ssem[s], add=True)

        def drain(sem, s):
            pltpu.make_async_copy(h3_hbm.at[pl.ds(0, cw)], msg_v.at[s],
                                  sem).wait()

        def run_worker(row0, rows):
            n_chunks = rows // CHUNK_ROWS
            pltpu.sync_copy(idx_hbm.at[pl.ds(2 * row0, 2 * rows)],
                            idx_v.at[pl.ds(0, 2 * rows)])
            issue_gathers(0, 0)
            issue_gathers(1, 1)

            @pl.loop(0, (n_chunks - 2) // 2)
            def _pair(k):
                i0 = 2 * k
                drain(gsem[0], 0)
                issue_scatters(i0, 0)
                drain(ssem[0], 0)
                issue_gathers(i0 + 2, 0)
                drain(gsem[1], 1)
                issue_scatters(i0 + 1, 1)
                drain(ssem[1], 1)
                issue_gathers(i0 + 3, 1)

            drain(gsem[0], 0)
            issue_scatters(n_chunks - 2, 0)
            drain(gsem[1], 1)
            issue_scatters(n_chunks - 1, 1)
            drain(ssem[0], 0)
            drain(ssem[1], 1)

        zcp.wait()
        plsc.subcore_barrier()

        @pl.when(core == 0)
        def _sc0():
            run_worker(sub * r0_rows, r0_rows)

        @pl.when(core == 1)
        def _sc1():
            run_worker(NS * r0_rows + sub * r1_rows, r1_rows)

        plsc.subcore_barrier()
        pltpu.sync_copy(acc_sh.at[my_acc], out_hbm.at[core].at[my_acc])

    return edge_pass


def _make_cnt_pass(cacc_rows, rows_pad):
    rows_per_worker = rows_pad // NW
    n_chunks = rows_per_worker // CNT_CHUNK
    rows_per_sub = cacc_rows // NS

    @functools.partial(
        pl.kernel,
        out_type=jax.ShapeDtypeStruct((NC, cacc_rows, 16), jnp.float32),
        mesh=_edge_mesh(),
        scratch_types=[
            pltpu.VMEM((CNT_CHUNK, ROW), jnp.int32),
            pltpu.VMEM((ROW, 16), jnp.float32),
            pltpu.VMEM_SHARED((cacc_rows, 16), jnp.float32),
        ],
        compiler_params=pltpu.CompilerParams(use_tc_tiling_on_sc=False),
    )
    def cnt_pass(cidx_hbm, ones_hbm, zeros_hbm, out_hbm,
                 cidx_v, ones_v, acc_sh):
        core = lax.axis_index("c")
        sub = lax.axis_index("s")
        wid = core * NS + sub

        my_acc = pl.ds(sub * rows_per_sub, rows_per_sub)
        pltpu.sync_copy(zeros_hbm.at[my_acc], acc_sh.at[my_acc])
        pltpu.sync_copy(ones_hbm, ones_v)
        plsc.subcore_barrier()

        row0 = wid * rows_per_worker

        @pl.loop(0, n_chunks)
        def _chunk(i):
            r0 = row0 + i * CNT_CHUNK
            pltpu.sync_copy(cidx_hbm.at[pl.ds(r0, CNT_CHUNK)], cidx_v)
            for j in range(CNT_CHUNK):
                pltpu.sync_copy(ones_v, acc_sh.at[cidx_v.at[j]], add=True)

        plsc.subcore_barrier()
        pltpu.sync_copy(acc_sh.at[my_acc], out_hbm.at[core].at[my_acc])

    return cnt_pass


def _prep_body(etg_ref, src_ref, dsts_ref, etc_ref, dstc_ref,
               idx_ref, cidx_ref, *, n_nodes):
    rows = etg_ref.shape[0]
    gidx = etg_ref[...] * n_nodes + src_ref[...]
    idx_ref[...] = jnp.stack([gidx, dsts_ref[...]], axis=1).reshape(
        2 * rows, gidx.shape[1])
    cidx_ref[...] = etc_ref[...] * n_nodes + dstc_ref[...]


def _proj_body(x_ref, w_ref, out_ref):
    x = x_ref[...]
    for c in range(w_ref.shape[0]):
        out_ref[c] = jnp.dot(x, w_ref[c], preferred_element_type=jnp.float32)


def _make_scale_body(c, n_rel, n_nodes):
    def body(h_ref, gt_ref, o_ref):
        f = jax.nn.softmax(gt_ref[...], axis=-1)[c]
        h = h_ref[...]
        for r in range(n_rel):
            o_ref[pl.ds(r * n_nodes, n_nodes)] = f[r] * h
    return body


def _make_combine_body(c, n_rel):
    def body(p_ref, c0_ref, c1_ref, c2_ref, h_ref, gt_ref, wg_ref, bg_ref,
             o_ref):
        f = jax.nn.softmax(gt_ref[...], axis=-1)[c]
        p = p_ref[...]
        s = p[0] + p[1]
        c16 = (f[0] * c0_ref[...] + f[1] * c1_ref[...]
               + f[2] * c2_ref[...])
        deg = c16[0, :, 0:1] + c16[1, :, 0:1] + f[n_rel]
        h = h_ref[...]
        agg = (s + f[n_rel] * h) / deg
        o_ref[...] = jnp.maximum(
            jnp.dot(agg, wg_ref[...], preferred_element_type=jnp.float32)
            + bg_ref[...], 0.0)
    return body


def _head_body(x0_ref, x1_ref, w1_ref, b1_ref, w2_ref, b2_ref, y_ref):
    hid = x0_ref.shape[1]
    xw = (jnp.dot(x0_ref[...], w1_ref[pl.ds(0, hid)],
                  preferred_element_type=jnp.float32)
          + jnp.dot(x1_ref[...], w1_ref[pl.ds(hid, hid)],
                    preferred_element_type=jnp.float32))
    h = jnp.maximum(xw + b1_ref[...], 0.0)
    y_ref[...] = jnp.dot(h, w2_ref[...], preferred_element_type=jnp.float32) \
        + b2_ref[...]


def kernel(x, edge_index, etype, W_gcn, gt_weight, Wg, bg, W1, b1, W2, b2):
    n, in_dim = x.shape
    e = edge_index.shape[1]
    n_ch, _, hid = W_gcn.shape
    n_layer, _, n_rel_full = gt_weight.shape
    n_rel = n_rel_full - 1
    num_class = W2.shape[1]

    rows_e = e // ROW
    import math
    row_gran = math.lcm(NS * CHUNK_ROWS * 2, NW * CNT_CHUNK)
    rows_pad = ((rows_e + row_gran - 1) // row_gran) * row_gran
    pad_edges = rows_pad * ROW - e
    acc_rows = ((n + 16 + 127) // 128) * 128
    cacc_rows = ((n_rel * n + 16 + 127) // 128) * 128

    src = edge_index[0]
    dst = edge_index[1]
    padi = jnp.arange(pad_edges, dtype=jnp.int32)
    padk = padi % 16
    zpad = jnp.zeros((pad_edges,), jnp.int32)
    et32 = etype.astype(jnp.int32)
    src_p = jnp.concatenate([src, padi % n]).reshape(rows_pad, ROW)
    etg_p = jnp.concatenate([et32, zpad]).reshape(rows_pad, ROW)
    dsts_p = jnp.concatenate([dst, (padi % 96) + n]).reshape(rows_pad, ROW)
    etc_p = jnp.concatenate(
        [et32, jnp.full((pad_edges,), n_rel, jnp.int32)]).reshape(rows_pad, ROW)
    dstc_p = jnp.concatenate([dst, padi % 64]).reshape(rows_pad, ROW)
    zeros_acc = jnp.zeros((acc_rows, hid), jnp.float32)
    zeros_cnt = jnp.zeros((cacc_rows, 16), jnp.float32)
    ones_row = jnp.ones((ROW, 16), jnp.float32)

    idx_il, cidx = pl.pallas_call(
        functools.partial(_prep_body, n_nodes=n),
        out_shape=[jax.ShapeDtypeStruct((2 * rows_pad, ROW), jnp.int32),
                   jax.ShapeDtypeStruct((rows_pad, ROW), jnp.int32)],
    )(etg_p, src_p, dsts_p, etc_p, dstc_p)

    cnt_pass = _make_cnt_pass(cacc_rows, rows_pad)
    cnt_part = cnt_pass(cidx, ones_row, zeros_cnt)

    H0 = pl.pallas_call(
        _proj_body,
        out_shape=jax.ShapeDtypeStruct((n_ch, n, hid), jnp.float32),
    )(x, W_gcn)

    edge_pass = _make_edge_pass(acc_rows, hid, rows_pad)

    scale_calls = [
        pl.pallas_call(
            _make_scale_body(c, n_rel, n),
            out_shape=jax.ShapeDtypeStruct((n_rel * n, hid), jnp.float32),
        ) for c in range(n_ch)
    ]

    BLK = 2000
    assert n % BLK == 0 and n_rel == 3
    nb = n // BLK
    grid = (nb,)
    combine_calls = []
    for c in range(n_ch):
        combine_calls.append(pl.pallas_call(
            _make_combine_body(c, n_rel),
            grid=grid,
            in_specs=[
                pl.BlockSpec((NC, BLK, hid), lambda i: (0, i, 0)),
                pl.BlockSpec((NC, BLK, 16), lambda i: (0, i, 0)),
                pl.BlockSpec((NC, BLK, 16), lambda i, _nb=nb: (0, _nb + i, 0)),
                pl.BlockSpec((NC, BLK, 16),
                             lambda i, _nb=nb: (0, 2 * _nb + i, 0)),
                pl.BlockSpec((BLK, hid), lambda i: (i, 0)),
                pl.BlockSpec((n_ch, n_rel_full), lambda i: (0, 0)),
                pl.BlockSpec((hid, hid), lambda i: (0, 0)),
                pl.BlockSpec((1, hid), lambda i: (0, 0)),
            ],
            out_specs=pl.BlockSpec((BLK, hid), lambda i: (i, 0)),
            out_shape=jax.ShapeDtypeStruct((n, hid), jnp.float32),
        ))

    H = [H0[c] for c in range(n_ch)]
    for l in range(n_layer):
        newH = []
        for c in range(n_ch):
            h3 = scale_calls[c](H[c], gt_weight[l])
            part = edge_pass(h3, idx_il, zeros_acc)
            newH.append(combine_calls[c](
                part, cnt_part, cnt_part, cnt_part, H[c], gt_weight[l],
                Wg, bg.reshape(1, hid)))
        H = newH

    y = pl.pallas_call(
        _head_body,
        out_shape=jax.ShapeDtypeStruct((n, num_class), jnp.float32),
    )(H[0], H[1], W1, b1.reshape(1, hid), W2, b2.reshape(1, num_class))
    return y
